# baseline (speedup 1.0000x reference)
import functools

import jax
import jax.numpy as jnp
from jax import lax
from jax.experimental import pallas as pl
from jax.experimental.pallas import tpu as pltpu
from jax.experimental.pallas import tpu_sc as plsc

_NC = 2
_NS = 16
_LANES = 16
_NBUF = 2


def kernel(x, table):
    b, s = x.shape
    _, d = table.shape
    nw = _NC * _NS
    pos_per_w = s // nw
    ch = 32
    n_ch = pos_per_w // ch

    mesh = plsc.VectorSubcoreMesh(core_axis_name="c", subcore_axis_name="s")

    @functools.partial(
        pl.kernel,
        mesh=mesh,
        out_type=jax.ShapeDtypeStruct((b * s, d), table.dtype),
        compiler_params=pltpu.CompilerParams(needs_layout_passes=False),
        scratch_types=[
            pltpu.VMEM((_NBUF, ch, d), jnp.float32),
            pltpu.VMEM((_LANES, d), jnp.float32),
            pltpu.VMEM((b, pos_per_w), jnp.int32),
            pltpu.VMEM((_NBUF, ch), jnp.int32),
            pltpu.VMEM((_LANES,), jnp.int32),
            pltpu.SemaphoreType.DMA,
            pltpu.SemaphoreType.DMA,
            [pltpu.SemaphoreType.DMA] * _NBUF,
            [pltpu.SemaphoreType.DMA] * _NBUF,
        ],
    )
    def run(x_hbm, tab_hbm, out_hbm, rows_v, row0_v, x_v, gidx_v, sidx_v,
            ssem, fsem, rsems, wsems):
        wid = lax.axis_index("s") * _NC + lax.axis_index("c")
        base = wid * pos_per_w
        iota = lax.iota(jnp.int32, _LANES)

        stage = [
            pltpu.async_copy(x_hbm.at[:, pl.ds(base, pos_per_w)], x_v, ssem),
        ]

        reads = [None] * _NBUF
        writes = [None] * _NBUF

        def start_read(c):
            buf = c % _NBUF
            for k in range(ch // _LANES):
                gidx_v[buf, pl.ds(k * _LANES, _LANES)] = (
                    base + c * ch + 1 + k * _LANES) + iota
            reads[buf] = pltpu.async_copy(
                tab_hbm.at[gidx_v.at[buf]], rows_v.at[buf], rsems[buf])

        start_read(0)
        total = None
        for c in range(n_ch):
            buf = c % _NBUF
            reads[buf].wait()
            cbase = base + c * ch
            writes[buf] = [
                pltpu.async_copy(
                    rows_v.at[buf], out_hbm.at[pl.ds(bb * s + cbase, ch)],
                    wsems[buf])
                for bb in range(b)
            ]
            nxt = (c + 1) % _NBUF
            if writes[nxt] is not None:
                for w in writes[nxt]:
                    w.wait()
                writes[nxt] = None
            if c + 1 < n_ch:
                start_read(c + 1)
            if c == 0:
                for cp in stage:
                    cp.wait()
                acc = jnp.zeros((_LANES,), jnp.int32)
                for bb in range(b):
                    for g in range(pos_per_w // _LANES):
                        xv = x_v[bb, pl.ds(g * _LANES, _LANES)]
                        acc = acc + jnp.where(xv == 0, 1, 0)
                total = jnp.sum(acc)
        for w in writes[(n_ch - 1) % _NBUF]:
            w.wait()

        def fixup_groups():
          sidx_v[...] = jnp.zeros((_LANES,), jnp.int32)
          pltpu.async_copy(tab_hbm.at[sidx_v], row0_v, fsem).wait()
          for bb in range(b):
            def group_body(g, carry, bb=bb):
                off = g * _LANES
                xv = x_v[bb, pl.ds(off, _LANES)]
                zmask = xv == 0
                nz = jnp.sum(jnp.where(zmask, 1, 0))

                @pl.when(nz > 0)
                def _():
                    first = plsc.all_reduce_ffs(zmask)
                    rowbase = bb * s + base + off
                    sidx_v[...] = jnp.where(
                        zmask, rowbase + iota, rowbase + first)
                    pltpu.async_copy(
                        row0_v, out_hbm.at[sidx_v], fsem).wait()
                return carry

            lax.fori_loop(0, pos_per_w // _LANES, group_body, 0)

        pl.when(total > 0)(fixup_groups)

    return run(x, table).reshape(b, s, d)


# speedup vs baseline: 1.0129x; 1.0129x over previous
import functools

import jax
import jax.numpy as jnp
from jax import lax
from jax.experimental import pallas as pl
from jax.experimental.pallas import tpu as pltpu
from jax.experimental.pallas import tpu_sc as plsc

_NC = 2
_NS = 16
_LANES = 16
_NBUF = 3


def kernel(x, table):
    b, s = x.shape
    _, d = table.shape
    nw = _NC * _NS
    pos_per_w = s // nw
    ch = 32
    n_ch = pos_per_w // ch

    mesh = plsc.VectorSubcoreMesh(core_axis_name="c", subcore_axis_name="s")

    @functools.partial(
        pl.kernel,
        mesh=mesh,
        out_type=jax.ShapeDtypeStruct((b * s, d), table.dtype),
        compiler_params=pltpu.CompilerParams(needs_layout_passes=False),
        scratch_types=[
            pltpu.VMEM((_NBUF, ch, d), jnp.float32),
            pltpu.VMEM((_LANES, d), jnp.float32),
            pltpu.VMEM((b, pos_per_w), jnp.int32),
            pltpu.VMEM((_NBUF, ch), jnp.int32),
            pltpu.VMEM((_LANES,), jnp.int32),
            pltpu.SemaphoreType.DMA,
            pltpu.SemaphoreType.DMA,
            [pltpu.SemaphoreType.DMA] * _NBUF,
            [pltpu.SemaphoreType.DMA] * _NBUF,
        ],
    )
    def run(x_hbm, tab_hbm, out_hbm, rows_v, row0_v, x_v, gidx_v, sidx_v,
            ssem, fsem, rsems, wsems):
        wid = lax.axis_index("s") * _NC + lax.axis_index("c")
        base = wid * pos_per_w
        iota = lax.iota(jnp.int32, _LANES)

        stage = [
            pltpu.async_copy(x_hbm.at[:, pl.ds(base, pos_per_w)], x_v, ssem),
        ]

        reads = [None] * _NBUF
        writes = [None] * _NBUF

        def start_read(c):
            buf = c % _NBUF
            for k in range(ch // _LANES):
                gidx_v[buf, pl.ds(k * _LANES, _LANES)] = (
                    base + c * ch + 1 + k * _LANES) + iota
            reads[buf] = pltpu.async_copy(
                tab_hbm.at[gidx_v.at[buf]], rows_v.at[buf], rsems[buf])

        start_read(0)
        total = None
        for c in range(n_ch):
            buf = c % _NBUF
            reads[buf].wait()
            cbase = base + c * ch
            writes[buf] = [
                pltpu.async_copy(
                    rows_v.at[buf], out_hbm.at[pl.ds(bb * s + cbase, ch)],
                    wsems[buf])
                for bb in range(b)
            ]
            nxt = (c + 1) % _NBUF
            if writes[nxt] is not None:
                for w in writes[nxt]:
                    w.wait()
                writes[nxt] = None
            if c + 1 < n_ch:
                start_read(c + 1)
            if c == 0:
                for cp in stage:
                    cp.wait()
                acc = jnp.zeros((_LANES,), jnp.int32)
                for bb in range(b):
                    for g in range(pos_per_w // _LANES):
                        xv = x_v[bb, pl.ds(g * _LANES, _LANES)]
                        acc = acc + jnp.where(xv == 0, 1, 0)
                total = jnp.sum(acc)
        for w in writes[(n_ch - 1) % _NBUF]:
            w.wait()

        def fixup_groups():
          sidx_v[...] = jnp.zeros((_LANES,), jnp.int32)
          pltpu.async_copy(tab_hbm.at[sidx_v], row0_v, fsem).wait()
          for bb in range(b):
            def group_body(g, carry, bb=bb):
                off = g * _LANES
                xv = x_v[bb, pl.ds(off, _LANES)]
                zmask = xv == 0
                nz = jnp.sum(jnp.where(zmask, 1, 0))

                @pl.when(nz > 0)
                def _():
                    first = plsc.all_reduce_ffs(zmask)
                    rowbase = bb * s + base + off
                    sidx_v[...] = jnp.where(
                        zmask, rowbase + iota, rowbase + first)
                    pltpu.async_copy(
                        row0_v, out_hbm.at[sidx_v], fsem).wait()
                return carry

            lax.fori_loop(0, pos_per_w // _LANES, group_body, 0)

        pl.when(total > 0)(fixup_groups)

    return run(x, table).reshape(b, s, d)
